# Initial kernel scaffold; baseline (speedup 1.0000x reference)
import jax
import jax.numpy as jnp
from jax.experimental import pallas as pl


def kernel(x, edge_index, batch, W1, b1, W2, b2, W3, b3, Wfc, bfc):
    raise NotImplementedError("write your pallas kernel here")


# speedup vs baseline: 53.3804x; 53.3804x over previous
import functools

import jax
import jax.numpy as jnp
from jax import lax
from jax.experimental import pallas as pl
from jax.experimental.pallas import tpu as pltpu
from jax.experimental.pallas import tpu_sc as plsc

N_NODES = 10000
N_EDGES = 320000
D_FEAT = 128
HIDDEN = 16
NUM_GRAPHS = 64
NUM_CLASSES = 2

NP_ = 10240
NC = 2
NS = 16
NW = NC * NS
CH = 2000
NCHUNK = N_EDGES // (NW * CH)
ROWS_PER_TILE = NP_ // NS
NODES_PER_W = NP_ // NW
CNT_PAD = 80

_MESH = plsc.VectorSubcoreMesh(core_axis_name="c", subcore_axis_name="s",
                               num_cores=NC, num_subcores=NS)


def _hist_body(dst_hbm, batch_hbm, deg_out, cnt_out,
               acc_deg, acc_cnt, idxv, bidxv, ones_e, ones_b, stage, cbuf):
    c = lax.axis_index("c")
    s = lax.axis_index("s")
    wid = s * NC + c

    def fill_zero(i, _):
        stage[pl.ds(i * 16, 16)] = jnp.zeros((16,), jnp.float32)
        return 0
    lax.fori_loop(0, ROWS_PER_TILE // 16, fill_zero, 0)

    def fill_ones_e(i, _):
        ones_e[pl.ds(i * 16, 16)] = jnp.full((16,), 1.0, jnp.float32)
        return 0
    lax.fori_loop(0, CH // 16, fill_ones_e, 0)

    def fill_ones_b(i, _):
        ones_b[pl.ds(i * 16, 16)] = jnp.full((16,), 1.0, jnp.float32)
        return 0
    lax.fori_loop(0, NODES_PER_W // 16, fill_ones_b, 0)

    pltpu.sync_copy(stage, acc_deg.at[pl.ds(s * ROWS_PER_TILE, ROWS_PER_TILE)])

    @pl.when(s == 0)
    def _():
        pltpu.sync_copy(stage.at[pl.ds(0, CNT_PAD)], acc_cnt)

    plsc.subcore_barrier()

    for k in range(NCHUNK):
        base = pl.multiple_of((wid + NW * k) * CH, 8)
        pltpu.sync_copy(dst_hbm.at[pl.ds(base, CH)], idxv)
        pltpu.sync_copy(ones_e, acc_deg.at[idxv], add=True)

    bbase = pl.multiple_of(wid * NODES_PER_W, 8)
    pltpu.sync_copy(batch_hbm.at[pl.ds(bbase, NODES_PER_W)], bidxv)
    pltpu.sync_copy(ones_b, acc_cnt.at[bidxv], add=True)

    plsc.subcore_barrier()

    pltpu.sync_copy(acc_deg.at[pl.ds(s * ROWS_PER_TILE, ROWS_PER_TILE)], stage)
    pltpu.sync_copy(stage, deg_out.at[c, pl.ds(s * ROWS_PER_TILE, ROWS_PER_TILE)])

    @pl.when(s == 0)
    def _():
        pltpu.sync_copy(acc_cnt, cbuf)
        pltpu.sync_copy(cbuf, cnt_out.at[c])


_hist = pl.kernel(
    _hist_body,
    out_type=(jax.ShapeDtypeStruct((NC, NP_), jnp.float32),
              jax.ShapeDtypeStruct((NC, CNT_PAD), jnp.float32)),
    mesh=_MESH,
    scratch_types=[
        pltpu.VMEM_SHARED((NP_,), jnp.float32),
        pltpu.VMEM_SHARED((CNT_PAD,), jnp.float32),
        pltpu.VMEM((CH,), jnp.int32),
        pltpu.VMEM((NODES_PER_W,), jnp.int32),
        pltpu.VMEM((CH,), jnp.float32),
        pltpu.VMEM((NODES_PER_W,), jnp.float32),
        pltpu.VMEM((ROWS_PER_TILE,), jnp.float32),
        pltpu.VMEM((CNT_PAD,), jnp.float32),
    ],
    compiler_params=pltpu.CompilerParams(use_tc_tiling_on_sc=False),
)


def _scatter_body(y_hbm, src_hbm, dst_hbm, s_out,
                  acc, srcv, dstv, rows, stage, sem):
    c = lax.axis_index("c")
    s = lax.axis_index("s")
    wid = s * NC + c

    def fill_zero(i, _):
        stage[i] = jnp.zeros((16,), jnp.float32)
        return 0
    lax.fori_loop(0, ROWS_PER_TILE, fill_zero, 0)

    pltpu.sync_copy(stage, acc.at[pl.ds(s * ROWS_PER_TILE, ROWS_PER_TILE)])
    plsc.subcore_barrier()

    for k in range(NCHUNK):
        base = pl.multiple_of((wid + NW * k) * CH, 8)
        pltpu.sync_copy(src_hbm.at[pl.ds(base, CH)], srcv)
        pltpu.sync_copy(dst_hbm.at[pl.ds(base, CH)], dstv)
        pltpu.async_copy(y_hbm.at[srcv], rows, sem).wait()
        pltpu.sync_copy(rows, acc.at[dstv], add=True)

    plsc.subcore_barrier()

    pltpu.sync_copy(acc.at[pl.ds(s * ROWS_PER_TILE, ROWS_PER_TILE)], stage)
    pltpu.sync_copy(stage, s_out.at[c, pl.ds(s * ROWS_PER_TILE, ROWS_PER_TILE)])


_scatter = pl.kernel(
    _scatter_body,
    out_type=jax.ShapeDtypeStruct((NC, NP_, HIDDEN), jnp.float32),
    mesh=_MESH,
    scratch_types=[
        pltpu.VMEM_SHARED((NP_, HIDDEN), jnp.float32),
        pltpu.VMEM((CH,), jnp.int32),
        pltpu.VMEM((CH,), jnp.int32),
        pltpu.VMEM((CH, HIDDEN), jnp.float32),
        pltpu.VMEM((ROWS_PER_TILE, HIDDEN), jnp.float32),
        pltpu.SemaphoreType.DMA,
    ],
    compiler_params=pltpu.CompilerParams(use_tc_tiling_on_sc=False),
)


def _prep_body(x_ref, w_ref, degp_ref, y_out, dinv_out):
    degp = degp_ref[...]
    deg = degp[0] + degp[1] + 1.0
    dinv = lax.rsqrt(deg)
    dinv_out[...] = dinv
    xw = jnp.dot(x_ref[...], w_ref[...], preferred_element_type=jnp.float32)
    y_out[...] = xw * dinv[:, None]


_prep = pl.pallas_call(
    _prep_body,
    out_shape=(jax.ShapeDtypeStruct((NP_, HIDDEN), jnp.float32),
               jax.ShapeDtypeStruct((NP_,), jnp.float32)),
)


def _combine_body(sp_ref, y_ref, dinv_ref, b_ref, w_ref, yn_out):
    sp = sp_ref[...]
    t = sp[0] + sp[1] + y_ref[...]
    dinv = dinv_ref[...]
    h = jnp.maximum(t * dinv[:, None] + b_ref[...][None, :], 0.0)
    rows = lax.broadcasted_iota(jnp.int32, (NP_, 1), 0)
    h = jnp.where(rows < N_NODES, h, 0.0)
    yn_out[...] = jnp.dot(h, w_ref[...], preferred_element_type=jnp.float32) * dinv[:, None]


_combine = pl.pallas_call(
    _combine_body,
    out_shape=jax.ShapeDtypeStruct((NP_, HIDDEN), jnp.float32),
)


def _final_body(sp_ref, y_ref, dinv_ref, b_ref, batch_ref, cntp_ref,
                wfc_ref, bfc_ref, out_ref):
    sp = sp_ref[...]
    t = sp[0] + sp[1] + y_ref[...]
    dinv = dinv_ref[...]
    h = jnp.maximum(t * dinv[:, None] + b_ref[...][None, :], 0.0)
    onehot = (batch_ref[...][:, None]
              == lax.broadcasted_iota(jnp.int32, (NP_, NUM_GRAPHS), 1)
              ).astype(jnp.float32)
    pooled_sum = lax.dot_general(onehot, h, (((0,), (0,)), ((), ())),
                                 preferred_element_type=jnp.float32)
    cnt = cntp_ref[...]
    counts = cnt[0, :NUM_GRAPHS] + cnt[1, :NUM_GRAPHS]
    pooled = pooled_sum / jnp.maximum(counts, 1.0)[:, None]
    logits = jnp.dot(pooled, wfc_ref[...], preferred_element_type=jnp.float32)
    logits = logits + bfc_ref[...][None, :]
    m = jnp.max(logits, axis=1, keepdims=True)
    lse = jnp.log(jnp.sum(jnp.exp(logits - m), axis=1, keepdims=True)) + m
    out_ref[...] = logits - lse


_final = pl.pallas_call(
    _final_body,
    out_shape=jax.ShapeDtypeStruct((NUM_GRAPHS, NUM_CLASSES), jnp.float32),
)


def kernel(x, edge_index, batch, W1, b1, W2, b2, W3, b3, Wfc, bfc):
    src = edge_index[0]
    dst = edge_index[1]
    x_pad = jnp.pad(x, ((0, NP_ - N_NODES), (0, 0)))
    pad_ids = NUM_GRAPHS + (jnp.arange(NP_ - N_NODES, dtype=batch.dtype) % 16)
    batch_pad = jnp.concatenate([batch, pad_ids])

    degp, cntp = _hist(dst, batch_pad)
    y1, dinv = _prep(x_pad, W1, degp)
    s1 = _scatter(y1, src, dst)
    y2 = _combine(s1, y1, dinv, b1, W2)
    s2 = _scatter(y2, src, dst)
    y3 = _combine(s2, y2, dinv, b2, W3)
    s3 = _scatter(y3, src, dst)
    return _final(s3, y3, dinv, b3, batch_pad, cntp, Wfc, bfc)


# speedup vs baseline: 66.2119x; 1.2404x over previous
import functools

import jax
import jax.numpy as jnp
from jax import lax
from jax.experimental import pallas as pl
from jax.experimental.pallas import tpu as pltpu
from jax.experimental.pallas import tpu_sc as plsc

N_NODES = 10000
N_EDGES = 320000
D_FEAT = 128
HIDDEN = 16
NUM_GRAPHS = 64
NUM_CLASSES = 2

NP_ = 10240
NC = 2
NS = 16
NW = NC * NS
CH = 2000
NCHUNK = N_EDGES // (NW * CH)
ROWS_PER_TILE = NP_ // NS
NODES_PER_W = NP_ // NW
CNT_PAD = 80

_MESH = plsc.VectorSubcoreMesh(core_axis_name="c", subcore_axis_name="s",
                               num_cores=NC, num_subcores=NS)
_SC_PARAMS = pltpu.CompilerParams(use_tc_tiling_on_sc=False)


def _hist_body(dst_hbm, batch_hbm, deg_out, cnt_out,
               acc_deg, acc_cnt, idxv, bidxv, ones_e, ones_b, stage, cbuf):
    c = lax.axis_index("c")
    s = lax.axis_index("s")
    wid = s * NC + c

    def fill_zero(i, _):
        stage[pl.ds(i * 16, 16)] = jnp.zeros((16,), jnp.float32)
        return 0
    lax.fori_loop(0, ROWS_PER_TILE // 16, fill_zero, 0)

    def fill_ones_e(i, _):
        ones_e[pl.ds(i * 16, 16)] = jnp.full((16,), 1.0, jnp.float32)
        return 0
    lax.fori_loop(0, CH // 16, fill_ones_e, 0)

    def fill_ones_b(i, _):
        ones_b[pl.ds(i * 16, 16)] = jnp.full((16,), 1.0, jnp.float32)
        return 0
    lax.fori_loop(0, NODES_PER_W // 16, fill_ones_b, 0)

    pltpu.sync_copy(stage, acc_deg.at[pl.ds(s * ROWS_PER_TILE, ROWS_PER_TILE)])

    @pl.when(s == 0)
    def _():
        pltpu.sync_copy(stage.at[pl.ds(0, CNT_PAD)], acc_cnt)

    plsc.subcore_barrier()

    for k in range(NCHUNK):
        base = pl.multiple_of((wid + NW * k) * CH, 8)
        pltpu.sync_copy(dst_hbm.at[pl.ds(base, CH)], idxv)
        pltpu.sync_copy(ones_e, acc_deg.at[idxv], add=True)

    bbase = pl.multiple_of(wid * NODES_PER_W, 8)
    pltpu.sync_copy(batch_hbm.at[pl.ds(bbase, NODES_PER_W)], bidxv)
    pltpu.sync_copy(ones_b, acc_cnt.at[bidxv], add=True)

    plsc.subcore_barrier()

    pltpu.sync_copy(acc_deg.at[pl.ds(s * ROWS_PER_TILE, ROWS_PER_TILE)], stage)
    pltpu.sync_copy(
        stage, deg_out.at[pl.ds(c * NP_ + s * ROWS_PER_TILE, ROWS_PER_TILE)])

    @pl.when(s == 0)
    def _():
        pltpu.sync_copy(acc_cnt, cbuf)
        pltpu.sync_copy(cbuf, cnt_out.at[pl.ds(c * CNT_PAD, CNT_PAD)])


_hist = pl.kernel(
    _hist_body,
    out_type=(jax.ShapeDtypeStruct((NC * NP_,), jnp.float32),
              jax.ShapeDtypeStruct((NC * CNT_PAD,), jnp.float32)),
    mesh=_MESH,
    scratch_types=[
        pltpu.VMEM_SHARED((NP_,), jnp.float32),
        pltpu.VMEM_SHARED((CNT_PAD,), jnp.float32),
        pltpu.VMEM((CH,), jnp.int32),
        pltpu.VMEM((NODES_PER_W,), jnp.int32),
        pltpu.VMEM((CH,), jnp.float32),
        pltpu.VMEM((NODES_PER_W,), jnp.float32),
        pltpu.VMEM((ROWS_PER_TILE,), jnp.float32),
        pltpu.VMEM((CNT_PAD,), jnp.float32),
    ],
    compiler_params=_SC_PARAMS,
)


def _scatter_body(y_hbm, src_hbm, dst_hbm, s_out,
                  acc, srcv0, dstv0, rows0, srcv1, dstv1, rows1,
                  stage, sem0, sem1):
    c = lax.axis_index("c")
    s = lax.axis_index("s")
    wid = s * NC + c

    def fill_zero(i, _):
        stage[i] = jnp.zeros((16,), jnp.float32)
        return 0
    lax.fori_loop(0, ROWS_PER_TILE, fill_zero, 0)

    pltpu.sync_copy(stage, acc.at[pl.ds(s * ROWS_PER_TILE, ROWS_PER_TILE)])
    plsc.subcore_barrier()

    bufs = [(srcv0, dstv0, rows0, sem0), (srcv1, dstv1, rows1, sem1)]

    def chunk_base(k):
        return pl.multiple_of((wid + NW * k) * CH, 8)

    sv, dv, rv, sm = bufs[0]
    pltpu.sync_copy(src_hbm.at[pl.ds(chunk_base(0), CH)], sv)
    pltpu.sync_copy(dst_hbm.at[pl.ds(chunk_base(0), CH)], dv)
    descs = [pltpu.async_copy(y_hbm.at[sv], rv, sm)]

    for k in range(NCHUNK):
        sv, dv, rv, sm = bufs[k % 2]
        if k + 1 < NCHUNK:
            sv2, dv2, rv2, sm2 = bufs[(k + 1) % 2]
            pltpu.sync_copy(src_hbm.at[pl.ds(chunk_base(k + 1), CH)], sv2)
            pltpu.sync_copy(dst_hbm.at[pl.ds(chunk_base(k + 1), CH)], dv2)
            descs.append(pltpu.async_copy(y_hbm.at[sv2], rv2, sm2))
        descs[k].wait()
        pltpu.sync_copy(rv, acc.at[dv], add=True)

    plsc.subcore_barrier()

    pltpu.sync_copy(acc.at[pl.ds(s * ROWS_PER_TILE, ROWS_PER_TILE)], stage)
    pltpu.sync_copy(stage, s_out.at[c, pl.ds(s * ROWS_PER_TILE, ROWS_PER_TILE)])


_scatter = pl.kernel(
    _scatter_body,
    out_type=jax.ShapeDtypeStruct((NC, NP_, HIDDEN), jnp.float32),
    mesh=_MESH,
    scratch_types=[
        pltpu.VMEM_SHARED((NP_, HIDDEN), jnp.float32),
        pltpu.VMEM((CH,), jnp.int32),
        pltpu.VMEM((CH,), jnp.int32),
        pltpu.VMEM((CH, HIDDEN), jnp.float32),
        pltpu.VMEM((CH,), jnp.int32),
        pltpu.VMEM((CH,), jnp.int32),
        pltpu.VMEM((CH, HIDDEN), jnp.float32),
        pltpu.VMEM((ROWS_PER_TILE, HIDDEN), jnp.float32),
        pltpu.SemaphoreType.DMA,
        pltpu.SemaphoreType.DMA,
    ],
    compiler_params=_SC_PARAMS,
)


def _epilogue_rows(s_hbm, y_hbm, dinv_hbm, b_hbm, wid,
                   s0v, s1v, yv, dv, bv, node_fn):
    vbase = pl.multiple_of(wid * NODES_PER_W, 8)
    pltpu.sync_copy(s_hbm.at[0, pl.ds(vbase, NODES_PER_W)], s0v)
    pltpu.sync_copy(s_hbm.at[1, pl.ds(vbase, NODES_PER_W)], s1v)
    pltpu.sync_copy(y_hbm.at[pl.ds(vbase, NODES_PER_W)], yv)
    pltpu.sync_copy(dinv_hbm.at[pl.ds(vbase, NODES_PER_W)], dv)
    pltpu.sync_copy(b_hbm, bv)
    b = bv[...]

    def block(I, _):
        dvec = dv[pl.ds(I * 16, 16)]
        for j in range(16):
            i = I * 16 + j
            t = s0v[i] + s1v[i] + yv[i]
            h = jnp.maximum(t * dvec[j] + b, 0.0)
            node_fn(i, j, h, dvec)
        return 0
    lax.fori_loop(0, NODES_PER_W // 16, block, 0)


def _combine_body(s_hbm, y_hbm, dinv_hbm, b_hbm, w_hbm, yn_out,
                  s0v, s1v, yv, dv, bv, wv, houtv):
    c = lax.axis_index("c")
    s = lax.axis_index("s")
    wid = s * NC + c
    pltpu.sync_copy(w_hbm, wv)
    wrows = [wv[k] for k in range(HIDDEN)]

    def node_fn(i, j, h, dvec):
        o = h[0] * wrows[0]
        for k in range(1, HIDDEN):
            o = o + h[k] * wrows[k]
        houtv[i] = o * dvec[j]

    _epilogue_rows(s_hbm, y_hbm, dinv_hbm, b_hbm, wid,
                   s0v, s1v, yv, dv, bv, node_fn)
    vbase = pl.multiple_of(wid * NODES_PER_W, 8)
    pltpu.sync_copy(houtv, yn_out.at[pl.ds(vbase, NODES_PER_W)])


_combine = pl.kernel(
    _combine_body,
    out_type=jax.ShapeDtypeStruct((NP_, HIDDEN), jnp.float32),
    mesh=_MESH,
    scratch_types=[
        pltpu.VMEM((NODES_PER_W, HIDDEN), jnp.float32),
        pltpu.VMEM((NODES_PER_W, HIDDEN), jnp.float32),
        pltpu.VMEM((NODES_PER_W, HIDDEN), jnp.float32),
        pltpu.VMEM((NODES_PER_W,), jnp.float32),
        pltpu.VMEM((HIDDEN,), jnp.float32),
        pltpu.VMEM((HIDDEN, HIDDEN), jnp.float32),
        pltpu.VMEM((NODES_PER_W, HIDDEN), jnp.float32),
    ],
    compiler_params=_SC_PARAMS,
)


def _pool_body(s_hbm, y_hbm, dinv_hbm, b_hbm, batch_hbm, pool_out,
               acc, s0v, s1v, yv, dv, bv, bidxv, houtv, stage):
    c = lax.axis_index("c")
    s = lax.axis_index("s")
    wid = s * NC + c
    vbase = pl.multiple_of(wid * NODES_PER_W, 8)
    pltpu.sync_copy(batch_hbm.at[pl.ds(vbase, NODES_PER_W)], bidxv)

    def fill_zero(i, _):
        stage[i] = jnp.zeros((16,), jnp.float32)
        return 0
    lax.fori_loop(0, CNT_PAD // NS, fill_zero, 0)
    pltpu.sync_copy(stage, acc.at[pl.ds(s * (CNT_PAD // NS), CNT_PAD // NS)])
    plsc.subcore_barrier()

    def node_fn(i, j, h, dvec):
        houtv[i] = h

    _epilogue_rows(s_hbm, y_hbm, dinv_hbm, b_hbm, wid,
                   s0v, s1v, yv, dv, bv, node_fn)
    pltpu.sync_copy(houtv, acc.at[bidxv], add=True)
    plsc.subcore_barrier()

    @pl.when(s == 0)
    def _():
        pltpu.sync_copy(acc, houtv.at[pl.ds(0, CNT_PAD)])
        pltpu.sync_copy(houtv.at[pl.ds(0, CNT_PAD)], pool_out.at[c])


_pool = pl.kernel(
    _pool_body,
    out_type=jax.ShapeDtypeStruct((NC, CNT_PAD, HIDDEN), jnp.float32),
    mesh=_MESH,
    scratch_types=[
        pltpu.VMEM_SHARED((CNT_PAD, HIDDEN), jnp.float32),
        pltpu.VMEM((NODES_PER_W, HIDDEN), jnp.float32),
        pltpu.VMEM((NODES_PER_W, HIDDEN), jnp.float32),
        pltpu.VMEM((NODES_PER_W, HIDDEN), jnp.float32),
        pltpu.VMEM((NODES_PER_W,), jnp.float32),
        pltpu.VMEM((HIDDEN,), jnp.float32),
        pltpu.VMEM((NODES_PER_W,), jnp.int32),
        pltpu.VMEM((NODES_PER_W, HIDDEN), jnp.float32),
        pltpu.VMEM((CNT_PAD // NS, HIDDEN), jnp.float32),
    ],
    compiler_params=_SC_PARAMS,
)


def _mm1_body(x_ref, w_ref, xw_out):
    xw_out[...] = jnp.dot(x_ref[...], w_ref[...],
                          preferred_element_type=jnp.float32)


_mm1 = pl.pallas_call(
    _mm1_body,
    out_shape=jax.ShapeDtypeStruct((NP_, HIDDEN), jnp.float32),
)


def _scale_body(xw_ref, degp_ref, y_out, dinv_out):
    degp = degp_ref[...]
    deg = degp[:NP_] + degp[NP_:] + 1.0
    dinv = lax.rsqrt(deg)
    dinv_out[...] = dinv
    y_out[...] = xw_ref[...] * dinv[:, None]


_scale = pl.pallas_call(
    _scale_body,
    out_shape=(jax.ShapeDtypeStruct((NP_, HIDDEN), jnp.float32),
               jax.ShapeDtypeStruct((NP_,), jnp.float32)),
)


def _final_body(pp_ref, cnt_ref, wfc_ref, bfc_ref, out_ref):
    pp = pp_ref[...]
    psum = pp[0, :NUM_GRAPHS, :] + pp[1, :NUM_GRAPHS, :]
    cnt = cnt_ref[...]
    counts = cnt[:NUM_GRAPHS] + cnt[CNT_PAD:CNT_PAD + NUM_GRAPHS]
    pooled = psum / jnp.maximum(counts, 1.0)[:, None]
    logits = jnp.dot(pooled, wfc_ref[...], preferred_element_type=jnp.float32)
    logits = logits + bfc_ref[...][None, :]
    m = jnp.max(logits, axis=1, keepdims=True)
    lse = jnp.log(jnp.sum(jnp.exp(logits - m), axis=1, keepdims=True)) + m
    out_ref[...] = logits - lse


_final = pl.pallas_call(
    _final_body,
    out_shape=jax.ShapeDtypeStruct((NUM_GRAPHS, NUM_CLASSES), jnp.float32),
)


def kernel(x, edge_index, batch, W1, b1, W2, b2, W3, b3, Wfc, bfc):
    src = edge_index[0]
    dst = edge_index[1]
    x_pad = jnp.pad(x, ((0, NP_ - N_NODES), (0, 0)))
    pad_ids = NUM_GRAPHS + (jnp.arange(NP_ - N_NODES, dtype=batch.dtype) % 16)
    batch_pad = jnp.concatenate([batch, pad_ids])

    degp, cntp = _hist(dst, batch_pad)
    xw = _mm1(x_pad, W1)
    y1, dinv = _scale(xw, degp)
    s1 = _scatter(y1, src, dst)
    y2 = _combine(s1, y1, dinv, b1, W2)
    s2 = _scatter(y2, src, dst)
    y3 = _combine(s2, y2, dinv, b2, W3)
    s3 = _scatter(y3, src, dst)
    pp = _pool(s3, y3, dinv, b3, batch_pad)
    return _final(pp, cntp, Wfc, bfc)


# speedup vs baseline: 68.7511x; 1.0383x over previous
import functools

import jax
import jax.numpy as jnp
from jax import lax
from jax.experimental import pallas as pl
from jax.experimental.pallas import tpu as pltpu
from jax.experimental.pallas import tpu_sc as plsc

N_NODES = 10000
N_EDGES = 320000
D_FEAT = 128
HIDDEN = 16
NUM_GRAPHS = 64
NUM_CLASSES = 2

NP_ = 10240
NC = 2
NS = 16
NW = NC * NS
CH = 2000
NCHUNK = N_EDGES // (NW * CH)
ROWS_PER_TILE = NP_ // NS
NODES_PER_W = NP_ // NW
CNT_PAD = 80

_MESH = plsc.VectorSubcoreMesh(core_axis_name="c", subcore_axis_name="s",
                               num_cores=NC, num_subcores=NS)
_SC_PARAMS = pltpu.CompilerParams(use_tc_tiling_on_sc=False)


def _hist_body(dst_hbm, batch_hbm, deg_out, cnt_out,
               acc_deg, acc_cnt, idxv, bidxv, ones_e, ones_b, stage, cbuf):
    c = lax.axis_index("c")
    s = lax.axis_index("s")
    wid = s * NC + c

    def fill_zero(i, _):
        stage[pl.ds(i * 16, 16)] = jnp.zeros((16,), jnp.float32)
        return 0
    lax.fori_loop(0, ROWS_PER_TILE // 16, fill_zero, 0)

    def fill_ones_e(i, _):
        ones_e[pl.ds(i * 16, 16)] = jnp.full((16,), 1.0, jnp.float32)
        return 0
    lax.fori_loop(0, CH // 16, fill_ones_e, 0)

    def fill_ones_b(i, _):
        ones_b[pl.ds(i * 16, 16)] = jnp.full((16,), 1.0, jnp.float32)
        return 0
    lax.fori_loop(0, NODES_PER_W // 16, fill_ones_b, 0)

    pltpu.sync_copy(stage, acc_deg.at[pl.ds(s * ROWS_PER_TILE, ROWS_PER_TILE)])

    @pl.when(s == 0)
    def _():
        pltpu.sync_copy(stage.at[pl.ds(0, CNT_PAD)], acc_cnt)

    plsc.subcore_barrier()

    for k in range(NCHUNK):
        base = pl.multiple_of((wid + NW * k) * CH, 8)
        pltpu.sync_copy(dst_hbm.at[pl.ds(base, CH)], idxv)
        pltpu.sync_copy(ones_e, acc_deg.at[idxv], add=True)

    bbase = pl.multiple_of(wid * NODES_PER_W, 8)
    pltpu.sync_copy(batch_hbm.at[pl.ds(bbase, NODES_PER_W)], bidxv)
    pltpu.sync_copy(ones_b, acc_cnt.at[bidxv], add=True)

    plsc.subcore_barrier()

    pltpu.sync_copy(acc_deg.at[pl.ds(s * ROWS_PER_TILE, ROWS_PER_TILE)], stage)
    pltpu.sync_copy(
        stage, deg_out.at[pl.ds(c * NP_ + s * ROWS_PER_TILE, ROWS_PER_TILE)])

    @pl.when(s == 0)
    def _():
        pltpu.sync_copy(acc_cnt, cbuf)
        pltpu.sync_copy(cbuf, cnt_out.at[pl.ds(c * CNT_PAD, CNT_PAD)])


_hist = pl.kernel(
    _hist_body,
    out_type=(jax.ShapeDtypeStruct((NC * NP_,), jnp.float32),
              jax.ShapeDtypeStruct((NC * CNT_PAD,), jnp.float32)),
    mesh=_MESH,
    scratch_types=[
        pltpu.VMEM_SHARED((NP_,), jnp.float32),
        pltpu.VMEM_SHARED((CNT_PAD,), jnp.float32),
        pltpu.VMEM((CH,), jnp.int32),
        pltpu.VMEM((NODES_PER_W,), jnp.int32),
        pltpu.VMEM((CH,), jnp.float32),
        pltpu.VMEM((NODES_PER_W,), jnp.float32),
        pltpu.VMEM((ROWS_PER_TILE,), jnp.float32),
        pltpu.VMEM((CNT_PAD,), jnp.float32),
    ],
    compiler_params=_SC_PARAMS,
)


def _scatter_body(y_hbm, src_hbm, dst_hbm, s_out,
                  acc, sv0, sv1, sv2, sv3, sv4, dv0, dv1, dv2, dv3, dv4,
                  rows0, rows1, rows2,
                  isem0, isem1, gsem0, gsem1, gsem2, ssem0, ssem1, ssem2):
    c = lax.axis_index("c")
    s = lax.axis_index("s")
    wid = s * NC + c
    svs = [sv0, sv1, sv2, sv3, sv4]
    dvs = [dv0, dv1, dv2, dv3, dv4]
    rows = [rows0, rows1, rows2]
    gsems = [gsem0, gsem1, gsem2]
    ssems = [ssem0, ssem1, ssem2]

    def chunk_base(k):
        return pl.multiple_of((wid + NW * k) * CH, 8)

    idescs = []
    for k in range(NCHUNK):
        idescs.append(pltpu.async_copy(
            src_hbm.at[pl.ds(chunk_base(k), CH)], svs[k], isem0))
        idescs.append(pltpu.async_copy(
            dst_hbm.at[pl.ds(chunk_base(k), CH)], dvs[k], isem1))

    def fill_zero(i, _):
        rows0[i] = jnp.zeros((16,), jnp.float32)
        return 0
    lax.fori_loop(0, ROWS_PER_TILE, fill_zero, 0)
    pltpu.sync_copy(rows0.at[pl.ds(0, ROWS_PER_TILE)],
                    acc.at[pl.ds(s * ROWS_PER_TILE, ROWS_PER_TILE)])
    for d in idescs:
        d.wait()
    plsc.subcore_barrier()

    gd = {}
    sd = {}

    def start_gather(k):
        gd[k] = pltpu.async_copy(y_hbm.at[svs[k]], rows[k % 3], gsems[k % 3])

    start_gather(0)
    start_gather(1)
    start_gather(2)
    for k in range(NCHUNK):
        gd[k].wait()
        sd[k] = pltpu.async_copy(rows[k % 3], acc.at[dvs[k]], ssems[k % 3],
                                 add=True)
        if k + 3 < NCHUNK:
            sd[k].wait()
            start_gather(k + 3)
    for k in range(max(0, NCHUNK - 3), NCHUNK):
        sd[k].wait()

    plsc.subcore_barrier()

    pltpu.sync_copy(acc.at[pl.ds(s * ROWS_PER_TILE, ROWS_PER_TILE)],
                    rows0.at[pl.ds(0, ROWS_PER_TILE)])
    pltpu.sync_copy(rows0.at[pl.ds(0, ROWS_PER_TILE)],
                    s_out.at[c, pl.ds(s * ROWS_PER_TILE, ROWS_PER_TILE)])


_scatter = pl.kernel(
    _scatter_body,
    out_type=jax.ShapeDtypeStruct((NC, NP_, HIDDEN), jnp.float32),
    mesh=_MESH,
    scratch_types=[
        pltpu.VMEM_SHARED((NP_, HIDDEN), jnp.float32),
        pltpu.VMEM((CH,), jnp.int32),
        pltpu.VMEM((CH,), jnp.int32),
        pltpu.VMEM((CH,), jnp.int32),
        pltpu.VMEM((CH,), jnp.int32),
        pltpu.VMEM((CH,), jnp.int32),
        pltpu.VMEM((CH,), jnp.int32),
        pltpu.VMEM((CH,), jnp.int32),
        pltpu.VMEM((CH,), jnp.int32),
        pltpu.VMEM((CH,), jnp.int32),
        pltpu.VMEM((CH,), jnp.int32),
        pltpu.VMEM((CH, HIDDEN), jnp.float32),
        pltpu.VMEM((CH, HIDDEN), jnp.float32),
        pltpu.VMEM((CH, HIDDEN), jnp.float32),
        pltpu.SemaphoreType.DMA,
        pltpu.SemaphoreType.DMA,
        pltpu.SemaphoreType.DMA,
        pltpu.SemaphoreType.DMA,
        pltpu.SemaphoreType.DMA,
        pltpu.SemaphoreType.DMA,
        pltpu.SemaphoreType.DMA,
        pltpu.SemaphoreType.DMA,
    ],
    compiler_params=_SC_PARAMS,
)


def _epilogue_rows(s_hbm, y_hbm, dinv_hbm, b_hbm, wid,
                   s0v, s1v, yv, dv, bv, node_fn):
    vbase = pl.multiple_of(wid * NODES_PER_W, 8)
    pltpu.sync_copy(s_hbm.at[0, pl.ds(vbase, NODES_PER_W)], s0v)
    pltpu.sync_copy(s_hbm.at[1, pl.ds(vbase, NODES_PER_W)], s1v)
    pltpu.sync_copy(y_hbm.at[pl.ds(vbase, NODES_PER_W)], yv)
    pltpu.sync_copy(dinv_hbm.at[pl.ds(vbase, NODES_PER_W)], dv)
    pltpu.sync_copy(b_hbm, bv)
    b = bv[...]

    def block(I, _):
        dvec = dv[pl.ds(I * 16, 16)]
        for j in range(16):
            i = I * 16 + j
            t = s0v[i] + s1v[i] + yv[i]
            h = jnp.maximum(t * dvec[j] + b, 0.0)
            node_fn(i, j, h, dvec)
        return 0
    lax.fori_loop(0, NODES_PER_W // 16, block, 0)


def _combine_body(s_hbm, y_hbm, dinv_hbm, b_hbm, w_hbm, yn_out,
                  s0v, s1v, yv, dv, bv, wv, houtv):
    c = lax.axis_index("c")
    s = lax.axis_index("s")
    wid = s * NC + c
    pltpu.sync_copy(w_hbm, wv)
    wrows = [wv[k] for k in range(HIDDEN)]

    def node_fn(i, j, h, dvec):
        ps = [h[k] * wrows[k] for k in range(HIDDEN)]
        while len(ps) > 1:
            ps = [ps[a] + ps[a + 1] for a in range(0, len(ps), 2)]
        houtv[i] = ps[0] * dvec[j]

    _epilogue_rows(s_hbm, y_hbm, dinv_hbm, b_hbm, wid,
                   s0v, s1v, yv, dv, bv, node_fn)
    vbase = pl.multiple_of(wid * NODES_PER_W, 8)
    pltpu.sync_copy(houtv, yn_out.at[pl.ds(vbase, NODES_PER_W)])


_combine = pl.kernel(
    _combine_body,
    out_type=jax.ShapeDtypeStruct((NP_, HIDDEN), jnp.float32),
    mesh=_MESH,
    scratch_types=[
        pltpu.VMEM((NODES_PER_W, HIDDEN), jnp.float32),
        pltpu.VMEM((NODES_PER_W, HIDDEN), jnp.float32),
        pltpu.VMEM((NODES_PER_W, HIDDEN), jnp.float32),
        pltpu.VMEM((NODES_PER_W,), jnp.float32),
        pltpu.VMEM((HIDDEN,), jnp.float32),
        pltpu.VMEM((HIDDEN, HIDDEN), jnp.float32),
        pltpu.VMEM((NODES_PER_W, HIDDEN), jnp.float32),
    ],
    compiler_params=_SC_PARAMS,
)


def _pool_body(s_hbm, y_hbm, dinv_hbm, b_hbm, batch_hbm, pool_out,
               acc, s0v, s1v, yv, dv, bv, bidxv, houtv, stage):
    c = lax.axis_index("c")
    s = lax.axis_index("s")
    wid = s * NC + c
    vbase = pl.multiple_of(wid * NODES_PER_W, 8)
    pltpu.sync_copy(batch_hbm.at[pl.ds(vbase, NODES_PER_W)], bidxv)

    def fill_zero(i, _):
        stage[i] = jnp.zeros((16,), jnp.float32)
        return 0
    lax.fori_loop(0, CNT_PAD // NS, fill_zero, 0)
    pltpu.sync_copy(stage, acc.at[pl.ds(s * (CNT_PAD // NS), CNT_PAD // NS)])
    plsc.subcore_barrier()

    def node_fn(i, j, h, dvec):
        houtv[i] = h

    _epilogue_rows(s_hbm, y_hbm, dinv_hbm, b_hbm, wid,
                   s0v, s1v, yv, dv, bv, node_fn)
    pltpu.sync_copy(houtv, acc.at[bidxv], add=True)
    plsc.subcore_barrier()

    @pl.when(s == 0)
    def _():
        pltpu.sync_copy(acc, houtv.at[pl.ds(0, CNT_PAD)])
        pltpu.sync_copy(houtv.at[pl.ds(0, CNT_PAD)], pool_out.at[c])


_pool = pl.kernel(
    _pool_body,
    out_type=jax.ShapeDtypeStruct((NC, CNT_PAD, HIDDEN), jnp.float32),
    mesh=_MESH,
    scratch_types=[
        pltpu.VMEM_SHARED((CNT_PAD, HIDDEN), jnp.float32),
        pltpu.VMEM((NODES_PER_W, HIDDEN), jnp.float32),
        pltpu.VMEM((NODES_PER_W, HIDDEN), jnp.float32),
        pltpu.VMEM((NODES_PER_W, HIDDEN), jnp.float32),
        pltpu.VMEM((NODES_PER_W,), jnp.float32),
        pltpu.VMEM((HIDDEN,), jnp.float32),
        pltpu.VMEM((NODES_PER_W,), jnp.int32),
        pltpu.VMEM((NODES_PER_W, HIDDEN), jnp.float32),
        pltpu.VMEM((CNT_PAD // NS, HIDDEN), jnp.float32),
    ],
    compiler_params=_SC_PARAMS,
)


def _mm1_body(x_ref, w_ref, xw_out):
    xw_out[...] = jnp.dot(x_ref[...], w_ref[...],
                          preferred_element_type=jnp.float32)


_mm1 = pl.pallas_call(
    _mm1_body,
    out_shape=jax.ShapeDtypeStruct((NP_, HIDDEN), jnp.float32),
)


def _scale_body(xw_ref, degp_ref, y_out, dinv_out):
    degp = degp_ref[...]
    deg = degp[:NP_] + degp[NP_:] + 1.0
    dinv = lax.rsqrt(deg)
    dinv_out[...] = dinv
    y_out[...] = xw_ref[...] * dinv[:, None]


_scale = pl.pallas_call(
    _scale_body,
    out_shape=(jax.ShapeDtypeStruct((NP_, HIDDEN), jnp.float32),
               jax.ShapeDtypeStruct((NP_,), jnp.float32)),
)


def _final_body(pp_ref, cnt_ref, wfc_ref, bfc_ref, out_ref):
    pp = pp_ref[...]
    psum = pp[0, :NUM_GRAPHS, :] + pp[1, :NUM_GRAPHS, :]
    cnt = cnt_ref[...]
    counts = cnt[:NUM_GRAPHS] + cnt[CNT_PAD:CNT_PAD + NUM_GRAPHS]
    pooled = psum / jnp.maximum(counts, 1.0)[:, None]
    logits = jnp.dot(pooled, wfc_ref[...], preferred_element_type=jnp.float32)
    logits = logits + bfc_ref[...][None, :]
    m = jnp.max(logits, axis=1, keepdims=True)
    lse = jnp.log(jnp.sum(jnp.exp(logits - m), axis=1, keepdims=True)) + m
    out_ref[...] = logits - lse


_final = pl.pallas_call(
    _final_body,
    out_shape=jax.ShapeDtypeStruct((NUM_GRAPHS, NUM_CLASSES), jnp.float32),
)


def kernel(x, edge_index, batch, W1, b1, W2, b2, W3, b3, Wfc, bfc):
    src = edge_index[0]
    dst = edge_index[1]
    x_pad = jnp.pad(x, ((0, NP_ - N_NODES), (0, 0)))
    pad_ids = NUM_GRAPHS + (jnp.arange(NP_ - N_NODES, dtype=batch.dtype) % 16)
    batch_pad = jnp.concatenate([batch, pad_ids])

    degp, cntp = _hist(dst, batch_pad)
    xw = _mm1(x_pad, W1)
    y1, dinv = _scale(xw, degp)
    s1 = _scatter(y1, src, dst)
    y2 = _combine(s1, y1, dinv, b1, W2)
    s2 = _scatter(y2, src, dst)
    y3 = _combine(s2, y2, dinv, b2, W3)
    s3 = _scatter(y3, src, dst)
    pp = _pool(s3, y3, dinv, b3, batch_pad)
    return _final(pp, cntp, Wfc, bfc)


# speedup vs baseline: 69.4619x; 1.0103x over previous
import functools

import jax
import jax.numpy as jnp
from jax import lax
from jax.experimental import pallas as pl
from jax.experimental.pallas import tpu as pltpu
from jax.experimental.pallas import tpu_sc as plsc

N_NODES = 10000
N_EDGES = 320000
D_FEAT = 128
HIDDEN = 16
NUM_GRAPHS = 64
NUM_CLASSES = 2

NP_ = 10240
NC = 2
NS = 16
NW = NC * NS
CH = 2000
NCHUNK = N_EDGES // (NW * CH)
ROWS_PER_TILE = NP_ // NS
NODES_PER_W = NP_ // NW
CNT_PAD = 80

_MESH = plsc.VectorSubcoreMesh(core_axis_name="c", subcore_axis_name="s",
                               num_cores=NC, num_subcores=NS)
_SC_PARAMS = pltpu.CompilerParams(use_tc_tiling_on_sc=False)


def _hist_body(dst_hbm, batch_hbm, deg_out, cnt_out,
               acc_deg, acc_cnt, idxv, bidxv, ones_e, ones_b, stage, cbuf):
    c = lax.axis_index("c")
    s = lax.axis_index("s")
    wid = s * NC + c

    def fill_zero(i, _):
        stage[pl.ds(i * 16, 16)] = jnp.zeros((16,), jnp.float32)
        return 0
    lax.fori_loop(0, ROWS_PER_TILE // 16, fill_zero, 0)

    def fill_ones_e(i, _):
        ones_e[pl.ds(i * 16, 16)] = jnp.full((16,), 1.0, jnp.float32)
        return 0
    lax.fori_loop(0, CH // 16, fill_ones_e, 0)

    def fill_ones_b(i, _):
        ones_b[pl.ds(i * 16, 16)] = jnp.full((16,), 1.0, jnp.float32)
        return 0
    lax.fori_loop(0, NODES_PER_W // 16, fill_ones_b, 0)

    pltpu.sync_copy(stage, acc_deg.at[pl.ds(s * ROWS_PER_TILE, ROWS_PER_TILE)])

    @pl.when(s == 0)
    def _():
        pltpu.sync_copy(stage.at[pl.ds(0, CNT_PAD)], acc_cnt)

    plsc.subcore_barrier()

    for k in range(NCHUNK):
        base = pl.multiple_of((wid + NW * k) * CH, 8)
        pltpu.sync_copy(dst_hbm.at[pl.ds(base, CH)], idxv)
        pltpu.sync_copy(ones_e, acc_deg.at[idxv], add=True)

    bbase = pl.multiple_of(wid * NODES_PER_W, 8)
    pltpu.sync_copy(batch_hbm.at[pl.ds(bbase, NODES_PER_W)], bidxv)
    pltpu.sync_copy(ones_b, acc_cnt.at[bidxv], add=True)

    plsc.subcore_barrier()

    pltpu.sync_copy(acc_deg.at[pl.ds(s * ROWS_PER_TILE, ROWS_PER_TILE)], stage)
    pltpu.sync_copy(
        stage, deg_out.at[pl.ds(c * NP_ + s * ROWS_PER_TILE, ROWS_PER_TILE)])

    @pl.when(s == 0)
    def _():
        pltpu.sync_copy(acc_cnt, cbuf)
        pltpu.sync_copy(cbuf, cnt_out.at[pl.ds(c * CNT_PAD, CNT_PAD)])


_hist = pl.kernel(
    _hist_body,
    out_type=(jax.ShapeDtypeStruct((NC * NP_,), jnp.float32),
              jax.ShapeDtypeStruct((NC * CNT_PAD,), jnp.float32)),
    mesh=_MESH,
    scratch_types=[
        pltpu.VMEM_SHARED((NP_,), jnp.float32),
        pltpu.VMEM_SHARED((CNT_PAD,), jnp.float32),
        pltpu.VMEM((CH,), jnp.int32),
        pltpu.VMEM((NODES_PER_W,), jnp.int32),
        pltpu.VMEM((CH,), jnp.float32),
        pltpu.VMEM((NODES_PER_W,), jnp.float32),
        pltpu.VMEM((ROWS_PER_TILE,), jnp.float32),
        pltpu.VMEM((CNT_PAD,), jnp.float32),
    ],
    compiler_params=_SC_PARAMS,
)


def _scatter_body(y_hbm, src_hbm, dst_hbm, s_out,
                  acc, sv0, sv1, sv2, sv3, sv4, dv0, dv1, dv2, dv3, dv4,
                  rows0, rows1, rows2,
                  isem0, isem1, gsem0, gsem1, gsem2, ssem0, ssem1, ssem2):
    c = lax.axis_index("c")
    s = lax.axis_index("s")
    wid = s * NC + c
    svs = [sv0, sv1, sv2, sv3, sv4]
    dvs = [dv0, dv1, dv2, dv3, dv4]
    rows = [rows0, rows1, rows2]
    gsems = [gsem0, gsem1, gsem2]
    ssems = [ssem0, ssem1, ssem2]

    def chunk_base(k):
        return pl.multiple_of((wid + NW * k) * CH, 8)

    idescs = []
    for k in range(NCHUNK):
        idescs.append(pltpu.async_copy(
            src_hbm.at[pl.ds(chunk_base(k), CH)], svs[k], isem0))
        idescs.append(pltpu.async_copy(
            dst_hbm.at[pl.ds(chunk_base(k), CH)], dvs[k], isem1))

    def fill_zero(i, _):
        rows0[i] = jnp.zeros((16,), jnp.float32)
        return 0
    lax.fori_loop(0, ROWS_PER_TILE, fill_zero, 0)
    pltpu.sync_copy(rows0.at[pl.ds(0, ROWS_PER_TILE)],
                    acc.at[pl.ds(s * ROWS_PER_TILE, ROWS_PER_TILE)])
    for d in idescs:
        d.wait()
    plsc.subcore_barrier()

    gd = {}
    sd = {}

    def start_gather(k):
        gd[k] = pltpu.async_copy(y_hbm.at[svs[k]], rows[k % 3], gsems[k % 3])

    start_gather(0)
    start_gather(1)
    start_gather(2)
    for k in range(NCHUNK):
        gd[k].wait()
        sd[k] = pltpu.async_copy(rows[k % 3], acc.at[dvs[k]], ssems[k % 3],
                                 add=True)
        if k + 3 < NCHUNK:
            sd[k].wait()
            start_gather(k + 3)
    for k in range(max(0, NCHUNK - 3), NCHUNK):
        sd[k].wait()

    plsc.subcore_barrier()

    pltpu.sync_copy(acc.at[pl.ds(s * ROWS_PER_TILE, ROWS_PER_TILE)],
                    rows0.at[pl.ds(0, ROWS_PER_TILE)])
    pltpu.sync_copy(rows0.at[pl.ds(0, ROWS_PER_TILE)],
                    s_out.at[c, pl.ds(s * ROWS_PER_TILE, ROWS_PER_TILE)])


_scatter = pl.kernel(
    _scatter_body,
    out_type=jax.ShapeDtypeStruct((NC, NP_, HIDDEN), jnp.float32),
    mesh=_MESH,
    scratch_types=[
        pltpu.VMEM_SHARED((NP_, HIDDEN), jnp.float32),
        pltpu.VMEM((CH,), jnp.int32),
        pltpu.VMEM((CH,), jnp.int32),
        pltpu.VMEM((CH,), jnp.int32),
        pltpu.VMEM((CH,), jnp.int32),
        pltpu.VMEM((CH,), jnp.int32),
        pltpu.VMEM((CH,), jnp.int32),
        pltpu.VMEM((CH,), jnp.int32),
        pltpu.VMEM((CH,), jnp.int32),
        pltpu.VMEM((CH,), jnp.int32),
        pltpu.VMEM((CH,), jnp.int32),
        pltpu.VMEM((CH, HIDDEN), jnp.float32),
        pltpu.VMEM((CH, HIDDEN), jnp.float32),
        pltpu.VMEM((CH, HIDDEN), jnp.float32),
        pltpu.SemaphoreType.DMA,
        pltpu.SemaphoreType.DMA,
        pltpu.SemaphoreType.DMA,
        pltpu.SemaphoreType.DMA,
        pltpu.SemaphoreType.DMA,
        pltpu.SemaphoreType.DMA,
        pltpu.SemaphoreType.DMA,
        pltpu.SemaphoreType.DMA,
    ],
    compiler_params=_SC_PARAMS,
)


def _epilogue_rows(s_hbm, y_hbm, dinv_hbm, b_hbm, wid,
                   s0v, s1v, yv, dv, bv, node_fn):
    vbase = pl.multiple_of(wid * NODES_PER_W, 8)
    pltpu.sync_copy(s_hbm.at[0, pl.ds(vbase, NODES_PER_W)], s0v)
    pltpu.sync_copy(s_hbm.at[1, pl.ds(vbase, NODES_PER_W)], s1v)
    pltpu.sync_copy(y_hbm.at[pl.ds(vbase, NODES_PER_W)], yv)
    pltpu.sync_copy(dinv_hbm.at[pl.ds(vbase, NODES_PER_W)], dv)
    pltpu.sync_copy(b_hbm, bv)
    b = bv[...]

    def block(I, _):
        dvec = dv[pl.ds(I * 16, 16)]
        for j in range(16):
            i = I * 16 + j
            t = s0v[i] + s1v[i] + yv[i]
            h = jnp.maximum(t * dvec[j] + b, 0.0)
            node_fn(i, j, h, dvec)
        return 0
    lax.fori_loop(0, NODES_PER_W // 16, block, 0)


def _combine_body(s_hbm, y_hbm, dinv_hbm, b_hbm, w_hbm, yn_out,
                  s0v, s1v, yv, dv, bv, wv, houtv):
    c = lax.axis_index("c")
    s = lax.axis_index("s")
    wid = s * NC + c
    pltpu.sync_copy(w_hbm, wv)
    wrows = [wv[k] for k in range(HIDDEN)]

    def node_fn(i, j, h, dvec):
        ps = [h[k] * wrows[k] for k in range(HIDDEN)]
        while len(ps) > 1:
            ps = [ps[a] + ps[a + 1] for a in range(0, len(ps), 2)]
        houtv[i] = ps[0] * dvec[j]

    _epilogue_rows(s_hbm, y_hbm, dinv_hbm, b_hbm, wid,
                   s0v, s1v, yv, dv, bv, node_fn)
    vbase = pl.multiple_of(wid * NODES_PER_W, 8)
    pltpu.sync_copy(houtv, yn_out.at[pl.ds(vbase, NODES_PER_W)])


_combine = pl.kernel(
    _combine_body,
    out_type=jax.ShapeDtypeStruct((NP_, HIDDEN), jnp.float32),
    mesh=_MESH,
    scratch_types=[
        pltpu.VMEM((NODES_PER_W, HIDDEN), jnp.float32),
        pltpu.VMEM((NODES_PER_W, HIDDEN), jnp.float32),
        pltpu.VMEM((NODES_PER_W, HIDDEN), jnp.float32),
        pltpu.VMEM((NODES_PER_W,), jnp.float32),
        pltpu.VMEM((HIDDEN,), jnp.float32),
        pltpu.VMEM((HIDDEN, HIDDEN), jnp.float32),
        pltpu.VMEM((NODES_PER_W, HIDDEN), jnp.float32),
    ],
    compiler_params=_SC_PARAMS,
)


def _pool_body(s_hbm, y_hbm, dinv_hbm, b_hbm, batch_hbm, pool_out,
               acc, s0v, s1v, yv, dv, bv, bidxv, houtv, stage):
    c = lax.axis_index("c")
    s = lax.axis_index("s")
    wid = s * NC + c
    vbase = pl.multiple_of(wid * NODES_PER_W, 8)
    pltpu.sync_copy(batch_hbm.at[pl.ds(vbase, NODES_PER_W)], bidxv)

    def fill_zero(i, _):
        stage[i] = jnp.zeros((16,), jnp.float32)
        return 0
    lax.fori_loop(0, CNT_PAD // NS, fill_zero, 0)
    pltpu.sync_copy(stage, acc.at[pl.ds(s * (CNT_PAD // NS), CNT_PAD // NS)])
    plsc.subcore_barrier()

    def node_fn(i, j, h, dvec):
        houtv[i] = h

    _epilogue_rows(s_hbm, y_hbm, dinv_hbm, b_hbm, wid,
                   s0v, s1v, yv, dv, bv, node_fn)
    pltpu.sync_copy(houtv, acc.at[bidxv], add=True)
    plsc.subcore_barrier()

    @pl.when(s == 0)
    def _():
        pltpu.sync_copy(acc, houtv.at[pl.ds(0, CNT_PAD)])
        pltpu.sync_copy(houtv.at[pl.ds(0, CNT_PAD)], pool_out.at[c])


_pool = pl.kernel(
    _pool_body,
    out_type=jax.ShapeDtypeStruct((NC, CNT_PAD, HIDDEN), jnp.float32),
    mesh=_MESH,
    scratch_types=[
        pltpu.VMEM_SHARED((CNT_PAD, HIDDEN), jnp.float32),
        pltpu.VMEM((NODES_PER_W, HIDDEN), jnp.float32),
        pltpu.VMEM((NODES_PER_W, HIDDEN), jnp.float32),
        pltpu.VMEM((NODES_PER_W, HIDDEN), jnp.float32),
        pltpu.VMEM((NODES_PER_W,), jnp.float32),
        pltpu.VMEM((HIDDEN,), jnp.float32),
        pltpu.VMEM((NODES_PER_W,), jnp.int32),
        pltpu.VMEM((NODES_PER_W, HIDDEN), jnp.float32),
        pltpu.VMEM((CNT_PAD // NS, HIDDEN), jnp.float32),
    ],
    compiler_params=_SC_PARAMS,
)


def _mm1_body(x_ref, w_ref, xw_out):
    xw_out[...] = jnp.dot(x_ref[...], w_ref[...],
                          preferred_element_type=jnp.float32)


_mm1 = pl.pallas_call(
    _mm1_body,
    out_shape=jax.ShapeDtypeStruct((NP_, HIDDEN), jnp.float32),
)


def _rsqrt_nr(d):
    s = (d + 1.0) * 0.5
    for _ in range(14):
        s = (s + d / s) * 0.5
    return 1.0 / s


def _scale_body(xw_hbm, degp_hbm, y_out, dinv_out, xwv, d0v, d1v, dvv):
    c = lax.axis_index("c")
    s = lax.axis_index("s")
    wid = s * NC + c
    vbase = pl.multiple_of(wid * NODES_PER_W, 8)
    pltpu.sync_copy(xw_hbm.at[pl.ds(vbase, NODES_PER_W)], xwv)
    pltpu.sync_copy(degp_hbm.at[pl.ds(vbase, NODES_PER_W)], d0v)
    pltpu.sync_copy(degp_hbm.at[pl.ds(NP_ + vbase, NODES_PER_W)], d1v)

    def blk(I, _):
        deg = d0v[pl.ds(I * 16, 16)] + d1v[pl.ds(I * 16, 16)] + 1.0
        dinv = _rsqrt_nr(deg)
        dvv[pl.ds(I * 16, 16)] = dinv
        for j in range(16):
            i = I * 16 + j
            xwv[i] = xwv[i] * dinv[j]
        return 0
    lax.fori_loop(0, NODES_PER_W // 16, blk, 0)
    pltpu.sync_copy(dvv, dinv_out.at[pl.ds(vbase, NODES_PER_W)])
    pltpu.sync_copy(xwv, y_out.at[pl.ds(vbase, NODES_PER_W)])


_scale = pl.kernel(
    _scale_body,
    out_type=(jax.ShapeDtypeStruct((NP_, HIDDEN), jnp.float32),
              jax.ShapeDtypeStruct((NP_,), jnp.float32)),
    mesh=_MESH,
    scratch_types=[
        pltpu.VMEM((NODES_PER_W, HIDDEN), jnp.float32),
        pltpu.VMEM((NODES_PER_W,), jnp.float32),
        pltpu.VMEM((NODES_PER_W,), jnp.float32),
        pltpu.VMEM((NODES_PER_W,), jnp.float32),
    ],
    compiler_params=_SC_PARAMS,
)


def _final_body(pp_ref, cnt_ref, wfc_ref, bfc_ref, out_ref):
    pp = pp_ref[...]
    psum = pp[0, :NUM_GRAPHS, :] + pp[1, :NUM_GRAPHS, :]
    cnt = cnt_ref[...]
    counts = cnt[:NUM_GRAPHS] + cnt[CNT_PAD:CNT_PAD + NUM_GRAPHS]
    pooled = psum / jnp.maximum(counts, 1.0)[:, None]
    logits = jnp.dot(pooled, wfc_ref[...], preferred_element_type=jnp.float32)
    logits = logits + bfc_ref[...][None, :]
    m = jnp.max(logits, axis=1, keepdims=True)
    lse = jnp.log(jnp.sum(jnp.exp(logits - m), axis=1, keepdims=True)) + m
    out_ref[...] = logits - lse


_final = pl.pallas_call(
    _final_body,
    out_shape=jax.ShapeDtypeStruct((NUM_GRAPHS, NUM_CLASSES), jnp.float32),
)


def kernel(x, edge_index, batch, W1, b1, W2, b2, W3, b3, Wfc, bfc):
    src = edge_index[0]
    dst = edge_index[1]
    x_pad = jnp.pad(x, ((0, NP_ - N_NODES), (0, 0)))
    pad_ids = NUM_GRAPHS + (jnp.arange(NP_ - N_NODES, dtype=batch.dtype) % 16)
    batch_pad = jnp.concatenate([batch, pad_ids])

    degp, cntp = _hist(dst, batch_pad)
    xw = _mm1(x_pad, W1)
    y1, dinv = _scale(xw, degp)
    s1 = _scatter(y1, src, dst)
    y2 = _combine(s1, y1, dinv, b1, W2)
    s2 = _scatter(y2, src, dst)
    y3 = _combine(s2, y2, dinv, b2, W3)
    s3 = _scatter(y3, src, dst)
    pp = _pool(s3, y3, dinv, b3, batch_pad)
    return _final(pp, cntp, Wfc, bfc)


# speedup vs baseline: 72.7208x; 1.0469x over previous
import functools

import jax
import jax.numpy as jnp
from jax import lax
from jax.experimental import pallas as pl
from jax.experimental.pallas import tpu as pltpu
from jax.experimental.pallas import tpu_sc as plsc

N_NODES = 10000
N_EDGES = 320000
D_FEAT = 128
HIDDEN = 16
NUM_GRAPHS = 64
NUM_CLASSES = 2

NP_ = 10240
NC = 2
NS = 16
NW = NC * NS
CH = 2000
NCHUNK = N_EDGES // (NW * CH)
ROWS_PER_TILE = NP_ // NS
NODES_PER_W = NP_ // NW
CNT_PAD = 80

_MESH = plsc.VectorSubcoreMesh(core_axis_name="c", subcore_axis_name="s",
                               num_cores=NC, num_subcores=NS)
_SC_PARAMS = pltpu.CompilerParams(use_tc_tiling_on_sc=False)


CH2 = 2048
NFULL = N_EDGES // CH2
TAIL = N_EDGES - NFULL * CH2
BPW = NP_ // NS


def _hist_body(ei_hbm, batch_hbm, deg_out, cnt_out,
               acc_deg, acc_cnt, idxv, idxt, idx1d, idxt1, bidxv,
               ones_e, ones_b, stage, cbuf):
    c = lax.axis_index("c")
    s = lax.axis_index("s")
    wid = s * NC + c

    def fill_zero(i, _):
        stage[pl.ds(i * 16, 16)] = jnp.zeros((16,), jnp.float32)
        return 0
    lax.fori_loop(0, ROWS_PER_TILE // 16, fill_zero, 0)

    def fill_ones_e(i, _):
        ones_e[pl.ds(i * 16, 16)] = jnp.full((16,), 1.0, jnp.float32)
        return 0
    lax.fori_loop(0, CH2 // 16, fill_ones_e, 0)

    def fill_ones_b(i, _):
        ones_b[pl.ds(i * 16, 16)] = jnp.full((16,), 1.0, jnp.float32)
        return 0
    lax.fori_loop(0, BPW // 16, fill_ones_b, 0)
    for i in range(128 // 16):
        cbuf[pl.ds(i * 16, 16)] = jnp.zeros((16,), jnp.float32)

    pltpu.sync_copy(stage, acc_deg.at[pl.ds(s * ROWS_PER_TILE, ROWS_PER_TILE)])

    @pl.when(s == 0)
    def _():
        pltpu.sync_copy(stage.at[pl.ds(0, CNT_PAD)], acc_cnt)

    plsc.subcore_barrier()

    for k in range(5):
        cid = wid + NW * k

        @pl.when(cid < NFULL)
        def _():
            base = pl.multiple_of(cid * CH2, 128)
            pltpu.sync_copy(ei_hbm.at[:, pl.ds(base, CH2)], idxv)

            def cp(m, _):
                idx1d[pl.ds(m * 16, 16)] = idxv[1, pl.ds(m * 16, 16)]
                return 0
            lax.fori_loop(0, CH2 // 16, cp, 0)
            pltpu.sync_copy(ones_e, acc_deg.at[idx1d], add=True)

    @pl.when(wid == NW - 1)
    def _():
        pltpu.sync_copy(ei_hbm.at[:, pl.ds(NFULL * CH2, TAIL)], idxt)

        def cpt(m, _):
            idxt1[pl.ds(m * 16, 16)] = idxt[1, pl.ds(m * 16, 16)]
            return 0
        lax.fori_loop(0, TAIL // 16, cpt, 0)
        pltpu.sync_copy(ones_e.at[pl.ds(0, TAIL)], acc_deg.at[idxt1],
                        add=True)

    @pl.when(c == 0)
    def _():
        bbase = pl.multiple_of(s * BPW, 128)
        pltpu.sync_copy(batch_hbm.at[pl.ds(bbase, BPW)], bidxv)
        pltpu.sync_copy(ones_b, acc_cnt.at[bidxv], add=True)

    plsc.subcore_barrier()

    pltpu.sync_copy(acc_deg.at[pl.ds(s * ROWS_PER_TILE, ROWS_PER_TILE)], stage)
    pltpu.sync_copy(
        stage, deg_out.at[pl.ds(c * NP_ + s * ROWS_PER_TILE, ROWS_PER_TILE)])

    @pl.when(s == 0)
    def _():
        pltpu.sync_copy(acc_cnt, cbuf.at[pl.ds(0, CNT_PAD)])
        pltpu.sync_copy(cbuf, cnt_out.at[pl.ds(c * 128, 128)])


_hist = pl.kernel(
    _hist_body,
    out_type=(jax.ShapeDtypeStruct((NC * NP_,), jnp.float32),
              jax.ShapeDtypeStruct((NC * 128,), jnp.float32)),
    mesh=_MESH,
    scratch_types=[
        pltpu.VMEM_SHARED((NP_,), jnp.float32),
        pltpu.VMEM_SHARED((CNT_PAD,), jnp.float32),
        pltpu.VMEM((2, CH2), jnp.int32),
        pltpu.VMEM((2, TAIL), jnp.int32),
        pltpu.VMEM((CH2,), jnp.int32),
        pltpu.VMEM((TAIL,), jnp.int32),
        pltpu.VMEM((BPW,), jnp.int32),
        pltpu.VMEM((CH2,), jnp.float32),
        pltpu.VMEM((BPW,), jnp.float32),
        pltpu.VMEM((ROWS_PER_TILE,), jnp.float32),
        pltpu.VMEM((128,), jnp.float32),
    ],
)


def _scatter_body(y_hbm, src_hbm, dst_hbm, s_out,
                  acc, sv0, sv1, sv2, sv3, sv4, dv0, dv1, dv2, dv3, dv4,
                  rows0, rows1, rows2,
                  isem0, isem1, gsem0, gsem1, gsem2, ssem0, ssem1, ssem2):
    c = lax.axis_index("c")
    s = lax.axis_index("s")
    wid = s * NC + c
    svs = [sv0, sv1, sv2, sv3, sv4]
    dvs = [dv0, dv1, dv2, dv3, dv4]
    rows = [rows0, rows1, rows2]
    gsems = [gsem0, gsem1, gsem2]
    ssems = [ssem0, ssem1, ssem2]

    def chunk_base(k):
        return pl.multiple_of((wid + NW * k) * CH, 8)

    idescs = []
    for k in range(NCHUNK):
        idescs.append(pltpu.async_copy(
            src_hbm.at[pl.ds(chunk_base(k), CH)], svs[k], isem0))
        idescs.append(pltpu.async_copy(
            dst_hbm.at[pl.ds(chunk_base(k), CH)], dvs[k], isem1))

    def fill_zero(i, _):
        rows0[i] = jnp.zeros((16,), jnp.float32)
        return 0
    lax.fori_loop(0, ROWS_PER_TILE, fill_zero, 0)
    pltpu.sync_copy(rows0.at[pl.ds(0, ROWS_PER_TILE)],
                    acc.at[pl.ds(s * ROWS_PER_TILE, ROWS_PER_TILE)])
    for d in idescs:
        d.wait()
    plsc.subcore_barrier()

    gd = {}
    sd = {}

    def start_gather(k):
        gd[k] = pltpu.async_copy(y_hbm.at[svs[k]], rows[k % 3], gsems[k % 3])

    start_gather(0)
    start_gather(1)
    start_gather(2)
    for k in range(NCHUNK):
        gd[k].wait()
        sd[k] = pltpu.async_copy(rows[k % 3], acc.at[dvs[k]], ssems[k % 3],
                                 add=True)
        if k + 3 < NCHUNK:
            sd[k].wait()
            start_gather(k + 3)
    for k in range(max(0, NCHUNK - 3), NCHUNK):
        sd[k].wait()

    plsc.subcore_barrier()

    pltpu.sync_copy(acc.at[pl.ds(s * ROWS_PER_TILE, ROWS_PER_TILE)],
                    rows0.at[pl.ds(0, ROWS_PER_TILE)])
    pltpu.sync_copy(rows0.at[pl.ds(0, ROWS_PER_TILE)],
                    s_out.at[c, pl.ds(s * ROWS_PER_TILE, ROWS_PER_TILE)])


_scatter = pl.kernel(
    _scatter_body,
    out_type=jax.ShapeDtypeStruct((NC, NP_, HIDDEN), jnp.float32),
    mesh=_MESH,
    scratch_types=[
        pltpu.VMEM_SHARED((NP_, HIDDEN), jnp.float32),
        pltpu.VMEM((CH,), jnp.int32),
        pltpu.VMEM((CH,), jnp.int32),
        pltpu.VMEM((CH,), jnp.int32),
        pltpu.VMEM((CH,), jnp.int32),
        pltpu.VMEM((CH,), jnp.int32),
        pltpu.VMEM((CH,), jnp.int32),
        pltpu.VMEM((CH,), jnp.int32),
        pltpu.VMEM((CH,), jnp.int32),
        pltpu.VMEM((CH,), jnp.int32),
        pltpu.VMEM((CH,), jnp.int32),
        pltpu.VMEM((CH, HIDDEN), jnp.float32),
        pltpu.VMEM((CH, HIDDEN), jnp.float32),
        pltpu.VMEM((CH, HIDDEN), jnp.float32),
        pltpu.SemaphoreType.DMA,
        pltpu.SemaphoreType.DMA,
        pltpu.SemaphoreType.DMA,
        pltpu.SemaphoreType.DMA,
        pltpu.SemaphoreType.DMA,
        pltpu.SemaphoreType.DMA,
        pltpu.SemaphoreType.DMA,
        pltpu.SemaphoreType.DMA,
    ],
    compiler_params=_SC_PARAMS,
)


def _epilogue_rows(s_hbm, y_hbm, dinv_hbm, b_hbm, wid,
                   s0v, s1v, yv, dv, bv, node_fn):
    vbase = pl.multiple_of(wid * NODES_PER_W, 8)
    pltpu.sync_copy(s_hbm.at[0, pl.ds(vbase, NODES_PER_W)], s0v)
    pltpu.sync_copy(s_hbm.at[1, pl.ds(vbase, NODES_PER_W)], s1v)
    pltpu.sync_copy(y_hbm.at[pl.ds(vbase, NODES_PER_W)], yv)
    pltpu.sync_copy(dinv_hbm.at[pl.ds(vbase, NODES_PER_W)], dv)
    pltpu.sync_copy(b_hbm, bv)
    b = bv[...]

    def block(I, _):
        dvec = dv[pl.ds(I * 16, 16)]
        for j in range(16):
            i = I * 16 + j
            t = s0v[i] + s1v[i] + yv[i]
            h = jnp.maximum(t * dvec[j] + b, 0.0)
            node_fn(i, j, h, dvec)
        return 0
    lax.fori_loop(0, NODES_PER_W // 16, block, 0)


def _combine_body(s_hbm, y_hbm, dinv_hbm, b_hbm, w_hbm, yn_out,
                  s0v, s1v, yv, dv, bv, wv, houtv):
    c = lax.axis_index("c")
    s = lax.axis_index("s")
    wid = s * NC + c
    pltpu.sync_copy(w_hbm, wv)
    wrows = [wv[k] for k in range(HIDDEN)]

    def node_fn(i, j, h, dvec):
        ps = [h[k] * wrows[k] for k in range(HIDDEN)]
        while len(ps) > 1:
            ps = [ps[a] + ps[a + 1] for a in range(0, len(ps), 2)]
        houtv[i] = ps[0] * dvec[j]

    _epilogue_rows(s_hbm, y_hbm, dinv_hbm, b_hbm, wid,
                   s0v, s1v, yv, dv, bv, node_fn)
    vbase = pl.multiple_of(wid * NODES_PER_W, 8)
    pltpu.sync_copy(houtv, yn_out.at[pl.ds(vbase, NODES_PER_W)])


_combine = pl.kernel(
    _combine_body,
    out_type=jax.ShapeDtypeStruct((NP_, HIDDEN), jnp.float32),
    mesh=_MESH,
    scratch_types=[
        pltpu.VMEM((NODES_PER_W, HIDDEN), jnp.float32),
        pltpu.VMEM((NODES_PER_W, HIDDEN), jnp.float32),
        pltpu.VMEM((NODES_PER_W, HIDDEN), jnp.float32),
        pltpu.VMEM((NODES_PER_W,), jnp.float32),
        pltpu.VMEM((HIDDEN,), jnp.float32),
        pltpu.VMEM((HIDDEN, HIDDEN), jnp.float32),
        pltpu.VMEM((NODES_PER_W, HIDDEN), jnp.float32),
    ],
    compiler_params=_SC_PARAMS,
)


def _pool_body(s_hbm, y_hbm, dinv_hbm, b_hbm, batch_hbm, pool_out,
               acc, s0v, s1v, yv, dv, bv, bidxv, houtv, stage):
    c = lax.axis_index("c")
    s = lax.axis_index("s")
    wid = s * NC + c
    vbase = pl.multiple_of(wid * NODES_PER_W, 8)
    pltpu.sync_copy(batch_hbm.at[pl.ds(vbase, NODES_PER_W)], bidxv)

    def fill_zero(i, _):
        stage[i] = jnp.zeros((16,), jnp.float32)
        return 0
    lax.fori_loop(0, CNT_PAD // NS, fill_zero, 0)
    pltpu.sync_copy(stage, acc.at[pl.ds(s * (CNT_PAD // NS), CNT_PAD // NS)])
    plsc.subcore_barrier()

    def node_fn(i, j, h, dvec):
        houtv[i] = h

    _epilogue_rows(s_hbm, y_hbm, dinv_hbm, b_hbm, wid,
                   s0v, s1v, yv, dv, bv, node_fn)
    pltpu.sync_copy(houtv, acc.at[bidxv], add=True)
    plsc.subcore_barrier()

    @pl.when(s == 0)
    def _():
        pltpu.sync_copy(acc, houtv.at[pl.ds(0, CNT_PAD)])
        pltpu.sync_copy(houtv.at[pl.ds(0, CNT_PAD)], pool_out.at[c])


_pool = pl.kernel(
    _pool_body,
    out_type=jax.ShapeDtypeStruct((NC, CNT_PAD, HIDDEN), jnp.float32),
    mesh=_MESH,
    scratch_types=[
        pltpu.VMEM_SHARED((CNT_PAD, HIDDEN), jnp.float32),
        pltpu.VMEM((NODES_PER_W, HIDDEN), jnp.float32),
        pltpu.VMEM((NODES_PER_W, HIDDEN), jnp.float32),
        pltpu.VMEM((NODES_PER_W, HIDDEN), jnp.float32),
        pltpu.VMEM((NODES_PER_W,), jnp.float32),
        pltpu.VMEM((HIDDEN,), jnp.float32),
        pltpu.VMEM((NODES_PER_W,), jnp.int32),
        pltpu.VMEM((NODES_PER_W, HIDDEN), jnp.float32),
        pltpu.VMEM((CNT_PAD // NS, HIDDEN), jnp.float32),
    ],
    compiler_params=_SC_PARAMS,
)


def _mm1_body(x_ref, w_ref, xw_out):
    xw_out[...] = jnp.dot(x_ref[...], w_ref[...],
                          preferred_element_type=jnp.float32)


_mm1 = pl.pallas_call(
    _mm1_body,
    out_shape=jax.ShapeDtypeStruct((NP_, HIDDEN), jnp.float32),
)


def _rsqrt_nr(d):
    s = (d + 1.0) * 0.5
    for _ in range(14):
        s = (s + d / s) * 0.5
    return 1.0 / s


def _scale_body(xw_hbm, degp_hbm, y_out, dinv_out, xwv, d0v, d1v, dvv):
    c = lax.axis_index("c")
    s = lax.axis_index("s")
    wid = s * NC + c
    vbase = pl.multiple_of(wid * NODES_PER_W, 8)
    pltpu.sync_copy(xw_hbm.at[pl.ds(vbase, NODES_PER_W)], xwv)
    pltpu.sync_copy(degp_hbm.at[pl.ds(vbase, NODES_PER_W)], d0v)
    pltpu.sync_copy(degp_hbm.at[pl.ds(NP_ + vbase, NODES_PER_W)], d1v)

    def blk(I, _):
        deg = d0v[pl.ds(I * 16, 16)] + d1v[pl.ds(I * 16, 16)] + 1.0
        dinv = _rsqrt_nr(deg)
        dvv[pl.ds(I * 16, 16)] = dinv
        for j in range(16):
            i = I * 16 + j
            xwv[i] = xwv[i] * dinv[j]
        return 0
    lax.fori_loop(0, NODES_PER_W // 16, blk, 0)
    pltpu.sync_copy(dvv, dinv_out.at[pl.ds(vbase, NODES_PER_W)])
    pltpu.sync_copy(xwv, y_out.at[pl.ds(vbase, NODES_PER_W)])


_scale = pl.kernel(
    _scale_body,
    out_type=(jax.ShapeDtypeStruct((NP_, HIDDEN), jnp.float32),
              jax.ShapeDtypeStruct((NP_,), jnp.float32)),
    mesh=_MESH,
    scratch_types=[
        pltpu.VMEM((NODES_PER_W, HIDDEN), jnp.float32),
        pltpu.VMEM((NODES_PER_W,), jnp.float32),
        pltpu.VMEM((NODES_PER_W,), jnp.float32),
        pltpu.VMEM((NODES_PER_W,), jnp.float32),
    ],
    compiler_params=_SC_PARAMS,
)


def _final_body(pp_ref, cnt_ref, wfc_ref, bfc_ref, out_ref):
    pp = pp_ref[...]
    psum = pp[0, :NUM_GRAPHS, :] + pp[1, :NUM_GRAPHS, :]
    cnt = cnt_ref[...]
    counts = cnt[:NUM_GRAPHS] + cnt[128:128 + NUM_GRAPHS]
    pooled = psum / jnp.maximum(counts, 1.0)[:, None]
    logits = jnp.dot(pooled, wfc_ref[...], preferred_element_type=jnp.float32)
    logits = logits + bfc_ref[...][None, :]
    m = jnp.max(logits, axis=1, keepdims=True)
    lse = jnp.log(jnp.sum(jnp.exp(logits - m), axis=1, keepdims=True)) + m
    out_ref[...] = logits - lse


_final = pl.pallas_call(
    _final_body,
    out_shape=jax.ShapeDtypeStruct((NUM_GRAPHS, NUM_CLASSES), jnp.float32),
)


def kernel(x, edge_index, batch, W1, b1, W2, b2, W3, b3, Wfc, bfc):
    src = edge_index[0]
    dst = edge_index[1]
    x_pad = jnp.pad(x, ((0, NP_ - N_NODES), (0, 0)))
    pad_ids = NUM_GRAPHS + (jnp.arange(NP_ - N_NODES, dtype=batch.dtype) % 16)
    batch_pad = jnp.concatenate([batch, pad_ids])

    degp, cntp = _hist(edge_index, batch_pad)
    xw = _mm1(x_pad, W1)
    y1, dinv = _scale(xw, degp)
    s1 = _scatter(y1, src, dst)
    y2 = _combine(s1, y1, dinv, b1, W2)
    s2 = _scatter(y2, src, dst)
    y3 = _combine(s2, y2, dinv, b2, W3)
    s3 = _scatter(y3, src, dst)
    pp = _pool(s3, y3, dinv, b3, batch_pad)
    return _final(pp, cntp, Wfc, bfc)


# speedup vs baseline: 75.0427x; 1.0319x over previous
import functools

import jax
import jax.numpy as jnp
from jax import lax
from jax.experimental import pallas as pl
from jax.experimental.pallas import tpu as pltpu
from jax.experimental.pallas import tpu_sc as plsc

N_NODES = 10000
N_EDGES = 320000
D_FEAT = 128
HIDDEN = 16
NUM_GRAPHS = 64
NUM_CLASSES = 2

NP_ = 10240
NC = 2
NS = 16
NW = NC * NS
CH = 2000
NCHUNK = N_EDGES // (NW * CH)
ROWS_PER_TILE = NP_ // NS
NODES_PER_W = NP_ // NW
CNT_PAD = 80

_MESH = plsc.VectorSubcoreMesh(core_axis_name="c", subcore_axis_name="s",
                               num_cores=NC, num_subcores=NS)
_SC_PARAMS = pltpu.CompilerParams(use_tc_tiling_on_sc=False)


CH2 = 2048
NFULL = N_EDGES // CH2
TAIL = N_EDGES - NFULL * CH2
BPW = NP_ // NS


def _hist_body(ei_hbm, batch_hbm, deg_out, cnt_out, src_out, dst_out,
               acc_deg, acc_cnt, idxv, idxt, idx1d, idxt1, src1d, bidxv,
               ones_e, ones_b, stage, cbuf):
    c = lax.axis_index("c")
    s = lax.axis_index("s")
    wid = s * NC + c

    def fill_zero(i, _):
        stage[pl.ds(i * 16, 16)] = jnp.zeros((16,), jnp.float32)
        return 0
    lax.fori_loop(0, ROWS_PER_TILE // 16, fill_zero, 0)

    def fill_ones_e(i, _):
        ones_e[pl.ds(i * 16, 16)] = jnp.full((16,), 1.0, jnp.float32)
        return 0
    lax.fori_loop(0, CH2 // 16, fill_ones_e, 0)

    def fill_ones_b(i, _):
        ones_b[pl.ds(i * 16, 16)] = jnp.full((16,), 1.0, jnp.float32)
        return 0
    lax.fori_loop(0, BPW // 16, fill_ones_b, 0)
    for i in range(128 // 16):
        cbuf[pl.ds(i * 16, 16)] = jnp.zeros((16,), jnp.float32)

    pltpu.sync_copy(stage, acc_deg.at[pl.ds(s * ROWS_PER_TILE, ROWS_PER_TILE)])

    @pl.when(s == 0)
    def _():
        pltpu.sync_copy(stage.at[pl.ds(0, CNT_PAD)], acc_cnt)

    plsc.subcore_barrier()

    for k in range(5):
        cid = wid + NW * k

        @pl.when(cid < NFULL)
        def _():
            base = pl.multiple_of(cid * CH2, 128)
            pltpu.sync_copy(ei_hbm.at[:, pl.ds(base, CH2)], idxv)

            def cp(m, _):
                idx1d[pl.ds(m * 16, 16)] = idxv[1, pl.ds(m * 16, 16)]
                src1d[pl.ds(m * 16, 16)] = idxv[0, pl.ds(m * 16, 16)]
                return 0
            lax.fori_loop(0, CH2 // 16, cp, 0)
            pltpu.sync_copy(src1d, src_out.at[pl.ds(base, CH2)])
            pltpu.sync_copy(idx1d, dst_out.at[pl.ds(base, CH2)])
            pltpu.sync_copy(ones_e, acc_deg.at[idx1d], add=True)

    @pl.when(wid == NW - 1)
    def _():
        pltpu.sync_copy(ei_hbm.at[:, pl.ds(NFULL * CH2, TAIL)], idxt)

        def cpt(m, _):
            idxt1[pl.ds(m * 16, 16)] = idxt[1, pl.ds(m * 16, 16)]
            return 0
        lax.fori_loop(0, TAIL // 16, cpt, 0)
        pltpu.sync_copy(idxt1, dst_out.at[pl.ds(NFULL * CH2, TAIL)])
        pltpu.sync_copy(ones_e.at[pl.ds(0, TAIL)], acc_deg.at[idxt1],
                        add=True)

        def cpt2(m, _):
            idxt1[pl.ds(m * 16, 16)] = idxt[0, pl.ds(m * 16, 16)]
            return 0
        lax.fori_loop(0, TAIL // 16, cpt2, 0)
        pltpu.sync_copy(idxt1, src_out.at[pl.ds(NFULL * CH2, TAIL)])

    @pl.when(c == 0)
    def _():
        bbase = pl.multiple_of(s * BPW, 128)
        pltpu.sync_copy(batch_hbm.at[pl.ds(bbase, BPW)], bidxv)
        pltpu.sync_copy(ones_b, acc_cnt.at[bidxv], add=True)

    plsc.subcore_barrier()

    pltpu.sync_copy(acc_deg.at[pl.ds(s * ROWS_PER_TILE, ROWS_PER_TILE)], stage)
    pltpu.sync_copy(
        stage, deg_out.at[pl.ds(c * NP_ + s * ROWS_PER_TILE, ROWS_PER_TILE)])

    @pl.when(s == 0)
    def _():
        pltpu.sync_copy(acc_cnt, cbuf.at[pl.ds(0, CNT_PAD)])
        pltpu.sync_copy(cbuf, cnt_out.at[pl.ds(c * 128, 128)])


_hist = pl.kernel(
    _hist_body,
    out_type=(jax.ShapeDtypeStruct((NC * NP_,), jnp.float32),
              jax.ShapeDtypeStruct((NC * 128,), jnp.float32),
              jax.ShapeDtypeStruct((N_EDGES,), jnp.int32),
              jax.ShapeDtypeStruct((N_EDGES,), jnp.int32)),
    mesh=_MESH,
    scratch_types=[
        pltpu.VMEM_SHARED((NP_,), jnp.float32),
        pltpu.VMEM_SHARED((CNT_PAD,), jnp.float32),
        pltpu.VMEM((2, CH2), jnp.int32),
        pltpu.VMEM((2, TAIL), jnp.int32),
        pltpu.VMEM((CH2,), jnp.int32),
        pltpu.VMEM((TAIL,), jnp.int32),
        pltpu.VMEM((CH2,), jnp.int32),
        pltpu.VMEM((BPW,), jnp.int32),
        pltpu.VMEM((CH2,), jnp.float32),
        pltpu.VMEM((BPW,), jnp.float32),
        pltpu.VMEM((ROWS_PER_TILE,), jnp.float32),
        pltpu.VMEM((128,), jnp.float32),
    ],
)


def _scatter_body(y_hbm, src_hbm, dst_hbm, s_out,
                  acc, sv0, sv1, sv2, sv3, sv4, dv0, dv1, dv2, dv3, dv4,
                  rows0, rows1, rows2,
                  isem0, isem1, gsem0, gsem1, gsem2, ssem0, ssem1, ssem2):
    c = lax.axis_index("c")
    s = lax.axis_index("s")
    wid = s * NC + c
    svs = [sv0, sv1, sv2, sv3, sv4]
    dvs = [dv0, dv1, dv2, dv3, dv4]
    rows = [rows0, rows1, rows2]
    gsems = [gsem0, gsem1, gsem2]
    ssems = [ssem0, ssem1, ssem2]

    def chunk_base(k):
        return pl.multiple_of((wid + NW * k) * CH, 8)

    idescs = []
    for k in range(NCHUNK):
        idescs.append(pltpu.async_copy(
            src_hbm.at[pl.ds(chunk_base(k), CH)], svs[k], isem0))
        idescs.append(pltpu.async_copy(
            dst_hbm.at[pl.ds(chunk_base(k), CH)], dvs[k], isem1))

    def fill_zero(i, _):
        rows0[i] = jnp.zeros((16,), jnp.float32)
        return 0
    lax.fori_loop(0, ROWS_PER_TILE, fill_zero, 0)
    pltpu.sync_copy(rows0.at[pl.ds(0, ROWS_PER_TILE)],
                    acc.at[pl.ds(s * ROWS_PER_TILE, ROWS_PER_TILE)])
    for d in idescs:
        d.wait()
    plsc.subcore_barrier()

    gd = {}
    sd = {}

    def start_gather(k):
        gd[k] = pltpu.async_copy(y_hbm.at[svs[k]], rows[k % 3], gsems[k % 3])

    start_gather(0)
    start_gather(1)
    start_gather(2)
    for k in range(NCHUNK):
        gd[k].wait()
        sd[k] = pltpu.async_copy(rows[k % 3], acc.at[dvs[k]], ssems[k % 3],
                                 add=True)
        if k + 3 < NCHUNK:
            sd[k].wait()
            start_gather(k + 3)
    for k in range(max(0, NCHUNK - 3), NCHUNK):
        sd[k].wait()

    plsc.subcore_barrier()

    pltpu.sync_copy(acc.at[pl.ds(s * ROWS_PER_TILE, ROWS_PER_TILE)],
                    rows0.at[pl.ds(0, ROWS_PER_TILE)])
    pltpu.sync_copy(rows0.at[pl.ds(0, ROWS_PER_TILE)],
                    s_out.at[c, pl.ds(s * ROWS_PER_TILE, ROWS_PER_TILE)])


_scatter = pl.kernel(
    _scatter_body,
    out_type=jax.ShapeDtypeStruct((NC, NP_, HIDDEN), jnp.float32),
    mesh=_MESH,
    scratch_types=[
        pltpu.VMEM_SHARED((NP_, HIDDEN), jnp.float32),
        pltpu.VMEM((CH,), jnp.int32),
        pltpu.VMEM((CH,), jnp.int32),
        pltpu.VMEM((CH,), jnp.int32),
        pltpu.VMEM((CH,), jnp.int32),
        pltpu.VMEM((CH,), jnp.int32),
        pltpu.VMEM((CH,), jnp.int32),
        pltpu.VMEM((CH,), jnp.int32),
        pltpu.VMEM((CH,), jnp.int32),
        pltpu.VMEM((CH,), jnp.int32),
        pltpu.VMEM((CH,), jnp.int32),
        pltpu.VMEM((CH, HIDDEN), jnp.float32),
        pltpu.VMEM((CH, HIDDEN), jnp.float32),
        pltpu.VMEM((CH, HIDDEN), jnp.float32),
        pltpu.SemaphoreType.DMA,
        pltpu.SemaphoreType.DMA,
        pltpu.SemaphoreType.DMA,
        pltpu.SemaphoreType.DMA,
        pltpu.SemaphoreType.DMA,
        pltpu.SemaphoreType.DMA,
        pltpu.SemaphoreType.DMA,
        pltpu.SemaphoreType.DMA,
    ],
    compiler_params=_SC_PARAMS,
)


def _epilogue_rows(s_hbm, y_hbm, dinv_hbm, b_hbm, wid,
                   s0v, s1v, yv, dv, bv, node_fn):
    vbase = pl.multiple_of(wid * NODES_PER_W, 8)
    pltpu.sync_copy(s_hbm.at[0, pl.ds(vbase, NODES_PER_W)], s0v)
    pltpu.sync_copy(s_hbm.at[1, pl.ds(vbase, NODES_PER_W)], s1v)
    pltpu.sync_copy(y_hbm.at[pl.ds(vbase, NODES_PER_W)], yv)
    pltpu.sync_copy(dinv_hbm.at[pl.ds(vbase, NODES_PER_W)], dv)
    pltpu.sync_copy(b_hbm, bv)
    b = bv[...]

    def block(I, _):
        dvec = dv[pl.ds(I * 16, 16)]
        for j in range(16):
            i = I * 16 + j
            t = s0v[i] + s1v[i] + yv[i]
            h = jnp.maximum(t * dvec[j] + b, 0.0)
            node_fn(i, j, h, dvec)
        return 0
    lax.fori_loop(0, NODES_PER_W // 16, block, 0)


def _combine_body(s_hbm, y_hbm, dinv_hbm, b_hbm, w_hbm, yn_out,
                  s0v, s1v, yv, dv, bv, wv, houtv):
    c = lax.axis_index("c")
    s = lax.axis_index("s")
    wid = s * NC + c
    pltpu.sync_copy(w_hbm, wv)
    wrows = [wv[k] for k in range(HIDDEN)]

    def node_fn(i, j, h, dvec):
        ps = [h[k] * wrows[k] for k in range(HIDDEN)]
        while len(ps) > 1:
            ps = [ps[a] + ps[a + 1] for a in range(0, len(ps), 2)]
        houtv[i] = ps[0] * dvec[j]

    _epilogue_rows(s_hbm, y_hbm, dinv_hbm, b_hbm, wid,
                   s0v, s1v, yv, dv, bv, node_fn)
    vbase = pl.multiple_of(wid * NODES_PER_W, 8)
    pltpu.sync_copy(houtv, yn_out.at[pl.ds(vbase, NODES_PER_W)])


_combine = pl.kernel(
    _combine_body,
    out_type=jax.ShapeDtypeStruct((NP_, HIDDEN), jnp.float32),
    mesh=_MESH,
    scratch_types=[
        pltpu.VMEM((NODES_PER_W, HIDDEN), jnp.float32),
        pltpu.VMEM((NODES_PER_W, HIDDEN), jnp.float32),
        pltpu.VMEM((NODES_PER_W, HIDDEN), jnp.float32),
        pltpu.VMEM((NODES_PER_W,), jnp.float32),
        pltpu.VMEM((HIDDEN,), jnp.float32),
        pltpu.VMEM((HIDDEN, HIDDEN), jnp.float32),
        pltpu.VMEM((NODES_PER_W, HIDDEN), jnp.float32),
    ],
    compiler_params=_SC_PARAMS,
)


def _pool_body(s_hbm, y_hbm, dinv_hbm, b_hbm, batch_hbm, pool_out,
               acc, s0v, s1v, yv, dv, bv, bidxv, houtv, stage):
    c = lax.axis_index("c")
    s = lax.axis_index("s")
    wid = s * NC + c
    vbase = pl.multiple_of(wid * NODES_PER_W, 8)
    pltpu.sync_copy(batch_hbm.at[pl.ds(vbase, NODES_PER_W)], bidxv)

    def fill_zero(i, _):
        stage[i] = jnp.zeros((16,), jnp.float32)
        return 0
    lax.fori_loop(0, CNT_PAD // NS, fill_zero, 0)
    pltpu.sync_copy(stage, acc.at[pl.ds(s * (CNT_PAD // NS), CNT_PAD // NS)])
    plsc.subcore_barrier()

    def node_fn(i, j, h, dvec):
        houtv[i] = h

    _epilogue_rows(s_hbm, y_hbm, dinv_hbm, b_hbm, wid,
                   s0v, s1v, yv, dv, bv, node_fn)
    pltpu.sync_copy(houtv, acc.at[bidxv], add=True)
    plsc.subcore_barrier()

    @pl.when(s == 0)
    def _():
        pltpu.sync_copy(acc, houtv.at[pl.ds(0, CNT_PAD)])
        pltpu.sync_copy(houtv.at[pl.ds(0, CNT_PAD)], pool_out.at[c])


_pool = pl.kernel(
    _pool_body,
    out_type=jax.ShapeDtypeStruct((NC, CNT_PAD, HIDDEN), jnp.float32),
    mesh=_MESH,
    scratch_types=[
        pltpu.VMEM_SHARED((CNT_PAD, HIDDEN), jnp.float32),
        pltpu.VMEM((NODES_PER_W, HIDDEN), jnp.float32),
        pltpu.VMEM((NODES_PER_W, HIDDEN), jnp.float32),
        pltpu.VMEM((NODES_PER_W, HIDDEN), jnp.float32),
        pltpu.VMEM((NODES_PER_W,), jnp.float32),
        pltpu.VMEM((HIDDEN,), jnp.float32),
        pltpu.VMEM((NODES_PER_W,), jnp.int32),
        pltpu.VMEM((NODES_PER_W, HIDDEN), jnp.float32),
        pltpu.VMEM((CNT_PAD // NS, HIDDEN), jnp.float32),
    ],
    compiler_params=_SC_PARAMS,
)


def _mm1_body(x_ref, w_ref, xw_out):
    xw_out[...] = jnp.dot(x_ref[...], w_ref[...],
                          preferred_element_type=jnp.float32)


_mm1 = pl.pallas_call(
    _mm1_body,
    out_shape=jax.ShapeDtypeStruct((NP_, HIDDEN), jnp.float32),
)


def _rsqrt_nr(d):
    s = (d + 1.0) * 0.5
    for _ in range(14):
        s = (s + d / s) * 0.5
    return 1.0 / s


def _scale_body(xw_hbm, degp_hbm, y_out, dinv_out, xwv, d0v, d1v, dvv):
    c = lax.axis_index("c")
    s = lax.axis_index("s")
    wid = s * NC + c
    vbase = pl.multiple_of(wid * NODES_PER_W, 8)
    pltpu.sync_copy(xw_hbm.at[pl.ds(vbase, NODES_PER_W)], xwv)
    pltpu.sync_copy(degp_hbm.at[pl.ds(vbase, NODES_PER_W)], d0v)
    pltpu.sync_copy(degp_hbm.at[pl.ds(NP_ + vbase, NODES_PER_W)], d1v)

    def blk(I, _):
        deg = d0v[pl.ds(I * 16, 16)] + d1v[pl.ds(I * 16, 16)] + 1.0
        dinv = _rsqrt_nr(deg)
        dvv[pl.ds(I * 16, 16)] = dinv
        for j in range(16):
            i = I * 16 + j
            xwv[i] = xwv[i] * dinv[j]
        return 0
    lax.fori_loop(0, NODES_PER_W // 16, blk, 0)
    pltpu.sync_copy(dvv, dinv_out.at[pl.ds(vbase, NODES_PER_W)])
    pltpu.sync_copy(xwv, y_out.at[pl.ds(vbase, NODES_PER_W)])


_scale = pl.kernel(
    _scale_body,
    out_type=(jax.ShapeDtypeStruct((NP_, HIDDEN), jnp.float32),
              jax.ShapeDtypeStruct((NP_,), jnp.float32)),
    mesh=_MESH,
    scratch_types=[
        pltpu.VMEM((NODES_PER_W, HIDDEN), jnp.float32),
        pltpu.VMEM((NODES_PER_W,), jnp.float32),
        pltpu.VMEM((NODES_PER_W,), jnp.float32),
        pltpu.VMEM((NODES_PER_W,), jnp.float32),
    ],
    compiler_params=_SC_PARAMS,
)


def _final_body(pp_ref, cnt_ref, wfc_ref, bfc_ref, out_ref):
    pp = pp_ref[...]
    psum = pp[0, :NUM_GRAPHS, :] + pp[1, :NUM_GRAPHS, :]
    cnt = cnt_ref[...]
    counts = cnt[:NUM_GRAPHS] + cnt[128:128 + NUM_GRAPHS]
    pooled = psum / jnp.maximum(counts, 1.0)[:, None]
    logits = jnp.dot(pooled, wfc_ref[...], preferred_element_type=jnp.float32)
    logits = logits + bfc_ref[...][None, :]
    m = jnp.max(logits, axis=1, keepdims=True)
    lse = jnp.log(jnp.sum(jnp.exp(logits - m), axis=1, keepdims=True)) + m
    out_ref[...] = logits - lse


_final = pl.pallas_call(
    _final_body,
    out_shape=jax.ShapeDtypeStruct((NUM_GRAPHS, NUM_CLASSES), jnp.float32),
)


def kernel(x, edge_index, batch, W1, b1, W2, b2, W3, b3, Wfc, bfc):
    x_pad = jnp.pad(x, ((0, NP_ - N_NODES), (0, 0)))
    pad_ids = NUM_GRAPHS + (jnp.arange(NP_ - N_NODES, dtype=batch.dtype) % 16)
    batch_pad = jnp.concatenate([batch, pad_ids])

    degp, cntp, src, dst = _hist(edge_index, batch_pad)
    xw = _mm1(x_pad, W1)
    y1, dinv = _scale(xw, degp)
    s1 = _scatter(y1, src, dst)
    y2 = _combine(s1, y1, dinv, b1, W2)
    s2 = _scatter(y2, src, dst)
    y3 = _combine(s2, y2, dinv, b2, W3)
    s3 = _scatter(y3, src, dst)
    pp = _pool(s3, y3, dinv, b3, batch_pad)
    return _final(pp, cntp, Wfc, bfc)
